# baseline (speedup 1.0000x reference)
import functools

import jax
import jax.numpy as jnp
from jax import lax
from jax.experimental import pallas as pl
from jax.experimental.pallas import tpu as pltpu
from jax.experimental.pallas import tpu_sc as plsc

_N = 10000
_D = 128
_E = 320000
_NPAD = 10240
_NC = 2
_NS = 16
_NW = _NC * _NS
_K = 128
_NCHUNK = 80
_EPW = _NCHUNK * _K
_EPAD = _EPW * _NW
_GCH = 10
_NGRP = _NCHUNK // _GCH
_RB = 256
_GRID = _NPAD // _RB
_RPT = _NPAD // _NS

_MESH = plsc.VectorSubcoreMesh(core_axis_name="c", subcore_axis_name="s")


@functools.partial(
    pl.kernel,
    out_type=jax.ShapeDtypeStruct((_NC, _NPAD, _D), jnp.float32),
    mesh=_MESH,
    scratch_types=[
        pltpu.VMEM((_GCH, _K), jnp.int32),
        pltpu.VMEM((_GCH, _K), jnp.int32),
        pltpu.VMEM((_K, _D), jnp.float32),
        pltpu.VMEM((_K, _D), jnp.float32),
        pltpu.SemaphoreType.DMA,
        pltpu.SemaphoreType.DMA,
        pltpu.SemaphoreType.DMA,
        pltpu.SemaphoreType.DMA,
        pltpu.VMEM_SHARED((_NPAD, _D), jnp.float32),
    ],
)
def _seg_sum(feat_hbm, src_hbm, dst_hbm, out_hbm,
             sidx, didx, r0b, r1b, g0, g1, s0, s1, acc_sh):
    c = lax.axis_index("c")
    s = lax.axis_index("s")
    wid = s * _NC + c
    rr0 = s * _RPT

    def zrow_body(i, carry):
        r0b[i // 8, pl.ds((i % 8) * 16, 16)] = jnp.zeros((16,), jnp.float32)
        return carry

    lax.fori_loop(0, _K * 8, zrow_body, 0)
    for j in range(_RPT // _K):
        pltpu.sync_copy(r0b, acc_sh.at[pl.ds(rr0 + j * _K, _K)])
    plsc.subcore_barrier()

    rows = (r0b, r1b)
    gsem = (g0, g1)
    ssem = (s0, s1)

    for grp in range(_NGRP):
        pltpu.sync_copy(src_hbm.at[wid, grp], sidx)
        pltpu.sync_copy(dst_hbm.at[wid, grp], didx)

        def body(p, carry):
            i0 = p * 2
            for b in range(2):
                pltpu.async_copy(feat_hbm.at[sidx.at[i0 + b]], rows[b], gsem[b])
            for b in range(2):
                pltpu.make_async_copy(
                    feat_hbm.at[sidx.at[i0 + b]], rows[b], gsem[b]).wait()
                pltpu.async_copy(rows[b], acc_sh.at[didx.at[i0 + b]],
                                 ssem[b], add=True)
            for b in range(2):
                pltpu.make_async_copy(
                    rows[b], acc_sh.at[didx.at[i0 + b]], ssem[b]).wait()
            return carry

        lax.fori_loop(0, _GCH // 2, body, 0)

    plsc.subcore_barrier()
    pltpu.sync_copy(acc_sh.at[pl.ds(rr0, _RPT)], out_hbm.at[c, pl.ds(rr0, _RPT)])


def _any_body(m_ref, o_ref):
    o_ref[...] = jnp.max(m_ref[...])[None, None]


_HR = 16
_HG = _EPAD // _K // _HR


def _hist_body(d_ref, o_ref):
    i = pl.program_id(0)

    @pl.when(i == 0)
    def _():
        o_ref[...] = jnp.zeros_like(o_ref)

    iota = lax.broadcasted_iota(jnp.int32, (_K, _K), 0)
    acc = jnp.zeros((_K, _K), jnp.float32)
    for r in range(_HR):
        d = d_ref[pl.ds(r, 1), :]
        hi = jnp.broadcast_to(d >> 7, (_K, _K))
        lo = jnp.broadcast_to(d & 127, (_K, _K))
        ah = (hi == iota).astype(jnp.bfloat16)
        al = (lo == iota).astype(jnp.bfloat16)
        acc += lax.dot_general(ah, al, (((1,), (1,)), ((), ())),
                               preferred_element_type=jnp.float32)
    o_ref[...] += acc


def _prep_body(x_ref, fm_ref, o_ref):
    o_ref[...] = jnp.where(fm_ref[...] > 0.0, 0.0, x_ref[...])


def _layer1_body(xz_ref, p_ref, c_ref, agg_ref, tm_ref, use_ref,
                 ws_ref, wn_ref, b_ref, h_ref, cnt_ref):
    ssum = p_ref[0] + p_ref[1]
    cntc = jnp.maximum(c_ref[...], 1.0)
    mean = ssum / cntc
    h = jnp.dot(xz_ref[...], ws_ref[...], preferred_element_type=jnp.float32)
    h += jnp.dot(mean, wn_ref[...], preferred_element_type=jnp.float32)
    h += b_ref[...]
    pre = jnp.dot(agg_ref[...], wn_ref[...], preferred_element_type=jnp.float32)
    pre += b_ref[...]
    cond = jnp.logical_and(tm_ref[...] > 0.0, use_ref[0, 0] > 0.0)
    h = jnp.where(cond, pre, h)
    h_ref[...] = jnp.maximum(h, 0.0)
    cnt_ref[...] = cntc


def _layer2_body(h_ref, p_ref, cnt_ref, ws_ref, wn_ref, b_ref, o_ref):
    mean = (p_ref[0] + p_ref[1]) / cnt_ref[...]
    o = jnp.dot(h_ref[...], ws_ref[...], preferred_element_type=jnp.float32)
    o += jnp.dot(mean, wn_ref[...], preferred_element_type=jnp.float32)
    o_ref[...] = o + b_ref[...]


def kernel(x, edge_index, frontier_mask, aggregated_neighbors, target_mask,
           W1_self, W1_neigh, b1, W2_self, W2_neigh, b2):
    f32 = jnp.float32
    npd = _NPAD - _N
    x_p = jnp.pad(x, ((0, npd), (0, 0)))
    agg_p = jnp.pad(aggregated_neighbors, ((0, npd), (0, 0)))
    fm = jnp.pad(frontier_mask.astype(f32), (0, npd))
    tm = jnp.pad(target_mask.astype(f32), (0, npd))
    fm_col = fm.reshape(_NPAD, 1)
    tm_col = tm.reshape(_NPAD, 1)
    fm2d = fm.reshape(_NPAD // 128, 128)
    src = jnp.pad(edge_index[0], (0, _EPAD - _E)).reshape(_NW, _NGRP, _GCH, _K)
    dst_flat = jnp.pad(edge_index[1], (0, _EPAD - _E), constant_values=_N)
    dst = dst_flat.reshape(_NW, _NGRP, _GCH, _K)
    b1r = b1.reshape(1, _D)
    b2r = b2.reshape(1, _D)
    use = pl.pallas_call(
        _any_body,
        out_shape=jax.ShapeDtypeStruct((1, 1), f32),
    )(fm2d)

    xz = pl.pallas_call(
        _prep_body,
        grid=(_GRID,),
        in_specs=[pl.BlockSpec((_RB, _D), lambda i: (i, 0)),
                  pl.BlockSpec((_RB, 1), lambda i: (i, 0))],
        out_specs=pl.BlockSpec((_RB, _D), lambda i: (i, 0)),
        out_shape=jax.ShapeDtypeStruct((_NPAD, _D), f32),
    )(x_p, fm_col)

    hist = pl.pallas_call(
        _hist_body,
        grid=(_HG,),
        in_specs=[pl.BlockSpec((_HR, _K), lambda i: (i, 0))],
        out_specs=pl.BlockSpec((_K, _K), lambda i: (0, 0)),
        out_shape=jax.ShapeDtypeStruct((_K, _K), f32),
    )(dst_flat.reshape(_EPAD // _K, _K))
    cnt_col = hist.reshape(-1)[:_NPAD].reshape(_NPAD, 1)

    part1 = _seg_sum(xz, src, dst)

    h1, cnt = pl.pallas_call(
        _layer1_body,
        grid=(_GRID,),
        in_specs=[
            pl.BlockSpec((_RB, _D), lambda i: (i, 0)),
            pl.BlockSpec((2, _RB, _D), lambda i: (0, i, 0)),
            pl.BlockSpec((_RB, 1), lambda i: (i, 0)),
            pl.BlockSpec((_RB, _D), lambda i: (i, 0)),
            pl.BlockSpec((_RB, 1), lambda i: (i, 0)),
            pl.BlockSpec((1, 1), lambda i: (0, 0)),
            pl.BlockSpec((_D, _D), lambda i: (0, 0)),
            pl.BlockSpec((_D, _D), lambda i: (0, 0)),
            pl.BlockSpec((1, _D), lambda i: (0, 0)),
        ],
        out_specs=[pl.BlockSpec((_RB, _D), lambda i: (i, 0)),
                   pl.BlockSpec((_RB, 1), lambda i: (i, 0))],
        out_shape=[jax.ShapeDtypeStruct((_NPAD, _D), f32),
                   jax.ShapeDtypeStruct((_NPAD, 1), f32)],
    )(xz, part1, cnt_col, agg_p, tm_col, use, W1_self, W1_neigh, b1r)

    part2 = _seg_sum(h1, src, dst)

    out = pl.pallas_call(
        _layer2_body,
        grid=(_GRID,),
        in_specs=[
            pl.BlockSpec((_RB, _D), lambda i: (i, 0)),
            pl.BlockSpec((2, _RB, _D), lambda i: (0, i, 0)),
            pl.BlockSpec((_RB, 1), lambda i: (i, 0)),
            pl.BlockSpec((_D, _D), lambda i: (0, 0)),
            pl.BlockSpec((_D, _D), lambda i: (0, 0)),
            pl.BlockSpec((1, _D), lambda i: (0, 0)),
        ],
        out_specs=pl.BlockSpec((_RB, _D), lambda i: (i, 0)),
        out_shape=jax.ShapeDtypeStruct((_NPAD, _D), f32),
    )(h1, part2, cnt, W2_self, W2_neigh, b2r)

    return out[:_N]


# speedup vs baseline: 1.0484x; 1.0484x over previous
import functools

import jax
import jax.numpy as jnp
from jax import lax
from jax.experimental import pallas as pl
from jax.experimental.pallas import tpu as pltpu
from jax.experimental.pallas import tpu_sc as plsc

_N = 10000
_D = 128
_E = 320000
_NPAD = 10240
_NC = 2
_NS = 16
_NW = _NC * _NS
_K = 128
_NCHUNK = 80
_EPW = _NCHUNK * _K
_EPAD = _EPW * _NW
_GCH = 10
_NGRP = _NCHUNK // _GCH
_RB = 256
_GRID = _NPAD // _RB
_RPT = _NPAD // _NS

_MESH = plsc.VectorSubcoreMesh(core_axis_name="c", subcore_axis_name="s")


@functools.partial(
    pl.kernel,
    out_type=jax.ShapeDtypeStruct((_NC, _NPAD, _D), jnp.float32),
    mesh=_MESH,
    scratch_types=[
        pltpu.VMEM((_GCH, _K), jnp.int32),
        pltpu.VMEM((_GCH, _K), jnp.int32),
        pltpu.VMEM((_GCH, _K), jnp.int32),
        pltpu.VMEM((_K, _D), jnp.float32),
        pltpu.VMEM((_K, _D), jnp.float32),
        pltpu.SemaphoreType.DMA,
        pltpu.SemaphoreType.DMA,
        pltpu.VMEM_SHARED((_NPAD, _D), jnp.float32),
    ],
)
def _seg_sum(feat_hbm, src_hbm, dst_hbm, out_hbm,
             sidx, didx0, didx1, r0b, r1b, s0, s1, acc_sh):
    c = lax.axis_index("c")
    s = lax.axis_index("s")
    wid = s * _NC + c
    rr0 = s * _RPT

    def zrow_body(i, carry):
        r0b[i // 8, pl.ds((i % 8) * 16, 16)] = jnp.zeros((16,), jnp.float32)
        return carry

    lax.fori_loop(0, _K * 8, zrow_body, 0)
    for j in range(_RPT // _K):
        pltpu.sync_copy(r0b, acc_sh.at[pl.ds(rr0 + j * _K, _K)])
    plsc.subcore_barrier()

    rows = (r0b, r1b)
    ssem = (s0, s1)
    didxs = (didx0, didx1)

    for grp in range(_NGRP):
        di = didxs[grp % 2]
        pltpu.sync_copy(src_hbm.at[wid, grp], sidx)
        pltpu.sync_copy(dst_hbm.at[wid, grp], di)

        def body(p, carry, di=di, first=(grp == 0)):
            i0 = p * 2
            for b in range(2):
                wait = pltpu.make_async_copy(
                    rows[b], acc_sh.at[di.at[i0 + b]], ssem[b]).wait
                if first:
                    pl.when(p > 0)(wait)
                else:
                    wait()
                pltpu.sync_copy(feat_hbm.at[sidx.at[i0 + b]], rows[b])
                pltpu.async_copy(rows[b], acc_sh.at[di.at[i0 + b]],
                                 ssem[b], add=True)
            return carry

        lax.fori_loop(0, _GCH // 2, body, 0)

    dlast = didxs[(_NGRP - 1) % 2]
    for b in range(2):
        pltpu.make_async_copy(
            rows[b], acc_sh.at[dlast.at[_GCH - 2 + b]], ssem[b]).wait()
    plsc.subcore_barrier()
    pltpu.sync_copy(acc_sh.at[pl.ds(rr0, _RPT)], out_hbm.at[c, pl.ds(rr0, _RPT)])


def _any_body(m_ref, o_ref):
    o_ref[...] = jnp.max(m_ref[...])[None, None]


_HR = 16
_HG = _EPAD // _K // _HR


def _hist_body(d_ref, o_ref):
    i = pl.program_id(0)

    @pl.when(i == 0)
    def _():
        o_ref[...] = jnp.zeros_like(o_ref)

    iota = lax.broadcasted_iota(jnp.int32, (_K, _K), 0)
    acc = jnp.zeros((_K, _K), jnp.float32)
    for r in range(_HR):
        d = d_ref[pl.ds(r, 1), :]
        hi = jnp.broadcast_to(d >> 7, (_K, _K))
        lo = jnp.broadcast_to(d & 127, (_K, _K))
        ah = (hi == iota).astype(jnp.bfloat16)
        al = (lo == iota).astype(jnp.bfloat16)
        acc += lax.dot_general(ah, al, (((1,), (1,)), ((), ())),
                               preferred_element_type=jnp.float32)
    o_ref[...] += acc


def _prep_body(x_ref, fm_ref, o_ref):
    o_ref[...] = jnp.where(fm_ref[...] > 0.0, 0.0, x_ref[...])


def _layer1_body(xz_ref, p_ref, c_ref, agg_ref, tm_ref, use_ref,
                 ws_ref, wn_ref, b_ref, h_ref, cnt_ref):
    ssum = p_ref[0] + p_ref[1]
    cntc = jnp.maximum(c_ref[...], 1.0)
    mean = ssum / cntc
    h = jnp.dot(xz_ref[...], ws_ref[...], preferred_element_type=jnp.float32)
    h += jnp.dot(mean, wn_ref[...], preferred_element_type=jnp.float32)
    h += b_ref[...]
    pre = jnp.dot(agg_ref[...], wn_ref[...], preferred_element_type=jnp.float32)
    pre += b_ref[...]
    cond = jnp.logical_and(tm_ref[...] > 0.0, use_ref[0, 0] > 0.0)
    h = jnp.where(cond, pre, h)
    h_ref[...] = jnp.maximum(h, 0.0)
    cnt_ref[...] = cntc


def _layer2_body(h_ref, p_ref, cnt_ref, ws_ref, wn_ref, b_ref, o_ref):
    mean = (p_ref[0] + p_ref[1]) / cnt_ref[...]
    o = jnp.dot(h_ref[...], ws_ref[...], preferred_element_type=jnp.float32)
    o += jnp.dot(mean, wn_ref[...], preferred_element_type=jnp.float32)
    o_ref[...] = o + b_ref[...]


def kernel(x, edge_index, frontier_mask, aggregated_neighbors, target_mask,
           W1_self, W1_neigh, b1, W2_self, W2_neigh, b2):
    f32 = jnp.float32
    npd = _NPAD - _N
    x_p = jnp.pad(x, ((0, npd), (0, 0)))
    agg_p = jnp.pad(aggregated_neighbors, ((0, npd), (0, 0)))
    fm = jnp.pad(frontier_mask.astype(f32), (0, npd))
    tm = jnp.pad(target_mask.astype(f32), (0, npd))
    fm_col = fm.reshape(_NPAD, 1)
    tm_col = tm.reshape(_NPAD, 1)
    fm2d = fm.reshape(_NPAD // 128, 128)
    src = jnp.pad(edge_index[0], (0, _EPAD - _E)).reshape(_NW, _NGRP, _GCH, _K)
    dst_flat = jnp.pad(edge_index[1], (0, _EPAD - _E), constant_values=_N)
    dst = dst_flat.reshape(_NW, _NGRP, _GCH, _K)
    b1r = b1.reshape(1, _D)
    b2r = b2.reshape(1, _D)
    use = pl.pallas_call(
        _any_body,
        out_shape=jax.ShapeDtypeStruct((1, 1), f32),
    )(fm2d)

    xz = pl.pallas_call(
        _prep_body,
        grid=(_GRID,),
        in_specs=[pl.BlockSpec((_RB, _D), lambda i: (i, 0)),
                  pl.BlockSpec((_RB, 1), lambda i: (i, 0))],
        out_specs=pl.BlockSpec((_RB, _D), lambda i: (i, 0)),
        out_shape=jax.ShapeDtypeStruct((_NPAD, _D), f32),
    )(x_p, fm_col)

    hist = pl.pallas_call(
        _hist_body,
        grid=(_HG,),
        in_specs=[pl.BlockSpec((_HR, _K), lambda i: (i, 0))],
        out_specs=pl.BlockSpec((_K, _K), lambda i: (0, 0)),
        out_shape=jax.ShapeDtypeStruct((_K, _K), f32),
    )(dst_flat.reshape(_EPAD // _K, _K))
    cnt_col = hist.reshape(-1)[:_NPAD].reshape(_NPAD, 1)

    part1 = _seg_sum(xz, src, dst)

    h1, cnt = pl.pallas_call(
        _layer1_body,
        grid=(_GRID,),
        in_specs=[
            pl.BlockSpec((_RB, _D), lambda i: (i, 0)),
            pl.BlockSpec((2, _RB, _D), lambda i: (0, i, 0)),
            pl.BlockSpec((_RB, 1), lambda i: (i, 0)),
            pl.BlockSpec((_RB, _D), lambda i: (i, 0)),
            pl.BlockSpec((_RB, 1), lambda i: (i, 0)),
            pl.BlockSpec((1, 1), lambda i: (0, 0)),
            pl.BlockSpec((_D, _D), lambda i: (0, 0)),
            pl.BlockSpec((_D, _D), lambda i: (0, 0)),
            pl.BlockSpec((1, _D), lambda i: (0, 0)),
        ],
        out_specs=[pl.BlockSpec((_RB, _D), lambda i: (i, 0)),
                   pl.BlockSpec((_RB, 1), lambda i: (i, 0))],
        out_shape=[jax.ShapeDtypeStruct((_NPAD, _D), f32),
                   jax.ShapeDtypeStruct((_NPAD, 1), f32)],
    )(xz, part1, cnt_col, agg_p, tm_col, use, W1_self, W1_neigh, b1r)

    part2 = _seg_sum(h1, src, dst)

    out = pl.pallas_call(
        _layer2_body,
        grid=(_GRID,),
        in_specs=[
            pl.BlockSpec((_RB, _D), lambda i: (i, 0)),
            pl.BlockSpec((2, _RB, _D), lambda i: (0, i, 0)),
            pl.BlockSpec((_RB, 1), lambda i: (i, 0)),
            pl.BlockSpec((_D, _D), lambda i: (0, 0)),
            pl.BlockSpec((_D, _D), lambda i: (0, 0)),
            pl.BlockSpec((1, _D), lambda i: (0, 0)),
        ],
        out_specs=pl.BlockSpec((_RB, _D), lambda i: (i, 0)),
        out_shape=jax.ShapeDtypeStruct((_NPAD, _D), f32),
    )(h1, part2, cnt, W2_self, W2_neigh, b2r)

    return out[:_N]


# speedup vs baseline: 1.2606x; 1.2023x over previous
import functools

import jax
import jax.numpy as jnp
from jax import lax
from jax.experimental import pallas as pl
from jax.experimental.pallas import tpu as pltpu
from jax.experimental.pallas import tpu_sc as plsc

_N = 10000
_D = 128
_E = 320000
_NPAD = 10240
_NC = 2
_NS = 16
_NW = _NC * _NS
_K = 128
_NCHUNK = 80
_EPW = _NCHUNK * _K
_EPAD = _EPW * _NW
_GCH = 40
_NGRP = _NCHUNK // _GCH
_RB = 256
_GRID = _NPAD // _RB
_RPT = _NPAD // _NS

_MESH = plsc.VectorSubcoreMesh(core_axis_name="c", subcore_axis_name="s")


@functools.partial(
    pl.kernel,
    out_type=jax.ShapeDtypeStruct((_NC, _NPAD, _D), jnp.float32),
    mesh=_MESH,
    scratch_types=[
        pltpu.VMEM((_GCH, _K), jnp.int32),
        pltpu.VMEM((_GCH, _K), jnp.int32),
        pltpu.VMEM((_GCH, _K), jnp.int32),
        pltpu.VMEM((_K, _D), jnp.float32),
        pltpu.VMEM((_K, _D), jnp.float32),
        pltpu.SemaphoreType.DMA,
        pltpu.SemaphoreType.DMA,
        pltpu.VMEM_SHARED((_NPAD, _D), jnp.float32),
    ],
)
def _seg_sum(feat_hbm, src_hbm, dst_hbm, out_hbm,
             sidx, didx0, didx1, r0b, r1b, s0, s1, acc_sh):
    c = lax.axis_index("c")
    s = lax.axis_index("s")
    wid = s * _NC + c
    rr0 = s * _RPT

    def zrow_body(i, carry):
        r0b[i // 8, pl.ds((i % 8) * 16, 16)] = jnp.zeros((16,), jnp.float32)
        return carry

    lax.fori_loop(0, _K * 8, zrow_body, 0)
    for j in range(_RPT // _K):
        pltpu.sync_copy(r0b, acc_sh.at[pl.ds(rr0 + j * _K, _K)])
    plsc.subcore_barrier()

    rows = (r0b, r1b)
    ssem = (s0, s1)
    didxs = (didx0, didx1)

    for grp in range(_NGRP):
        di = didxs[grp % 2]
        pltpu.sync_copy(src_hbm.at[wid, grp], sidx)
        pltpu.sync_copy(dst_hbm.at[wid, grp], di)

        def body(p, carry, di=di, first=(grp == 0)):
            i0 = p * 2
            for b in range(2):
                wait = pltpu.make_async_copy(
                    rows[b], acc_sh.at[di.at[i0 + b]], ssem[b]).wait
                if first:
                    pl.when(p > 0)(wait)
                else:
                    wait()
                pltpu.sync_copy(feat_hbm.at[sidx.at[i0 + b]], rows[b])
                pltpu.async_copy(rows[b], acc_sh.at[di.at[i0 + b]],
                                 ssem[b], add=True)
            return carry

        lax.fori_loop(0, _GCH // 2, body, 0)

    dlast = didxs[(_NGRP - 1) % 2]
    for b in range(2):
        pltpu.make_async_copy(
            rows[b], acc_sh.at[dlast.at[_GCH - 2 + b]], ssem[b]).wait()
    plsc.subcore_barrier()
    pltpu.sync_copy(acc_sh.at[pl.ds(rr0, _RPT)], out_hbm.at[c, pl.ds(rr0, _RPT)])


def _any_body(m_ref, o_ref):
    o_ref[...] = jnp.max(m_ref[...])[None, None]


_HR = 16
_HG = _EPAD // _K // _HR


def _hist_body(d_ref, o_ref):
    i = pl.program_id(0)

    @pl.when(i == 0)
    def _():
        o_ref[...] = jnp.zeros_like(o_ref)

    iota = lax.broadcasted_iota(jnp.int32, (_K, _K), 0)
    acc = jnp.zeros((_K, _K), jnp.float32)
    for r in range(_HR):
        d = d_ref[pl.ds(r, 1), :]
        hi = jnp.broadcast_to(d >> 7, (_K, _K))
        lo = jnp.broadcast_to(d & 127, (_K, _K))
        ah = (hi == iota).astype(jnp.bfloat16)
        al = (lo == iota).astype(jnp.bfloat16)
        acc += lax.dot_general(ah, al, (((1,), (1,)), ((), ())),
                               preferred_element_type=jnp.float32)
    o_ref[...] += acc


def _prep_body(x_ref, fm_ref, o_ref):
    o_ref[...] = jnp.where(fm_ref[...] > 0.0, 0.0, x_ref[...])


def _layer1_body(xz_ref, p_ref, c_ref, agg_ref, tm_ref, use_ref,
                 ws_ref, wn_ref, b_ref, h_ref, cnt_ref):
    ssum = p_ref[0] + p_ref[1]
    cntc = jnp.maximum(c_ref[...], 1.0)
    mean = ssum / cntc
    h = jnp.dot(xz_ref[...], ws_ref[...], preferred_element_type=jnp.float32)
    h += jnp.dot(mean, wn_ref[...], preferred_element_type=jnp.float32)
    h += b_ref[...]
    pre = jnp.dot(agg_ref[...], wn_ref[...], preferred_element_type=jnp.float32)
    pre += b_ref[...]
    cond = jnp.logical_and(tm_ref[...] > 0.0, use_ref[0, 0] > 0.0)
    h = jnp.where(cond, pre, h)
    h_ref[...] = jnp.maximum(h, 0.0)
    cnt_ref[...] = cntc


def _layer2_body(h_ref, p_ref, cnt_ref, ws_ref, wn_ref, b_ref, o_ref):
    mean = (p_ref[0] + p_ref[1]) / cnt_ref[...]
    o = jnp.dot(h_ref[...], ws_ref[...], preferred_element_type=jnp.float32)
    o += jnp.dot(mean, wn_ref[...], preferred_element_type=jnp.float32)
    o_ref[...] = o + b_ref[...]


def kernel(x, edge_index, frontier_mask, aggregated_neighbors, target_mask,
           W1_self, W1_neigh, b1, W2_self, W2_neigh, b2):
    f32 = jnp.float32
    npd = _NPAD - _N
    x_p = jnp.pad(x, ((0, npd), (0, 0)))
    agg_p = jnp.pad(aggregated_neighbors, ((0, npd), (0, 0)))
    fm = jnp.pad(frontier_mask.astype(f32), (0, npd))
    tm = jnp.pad(target_mask.astype(f32), (0, npd))
    fm_col = fm.reshape(_NPAD, 1)
    tm_col = tm.reshape(_NPAD, 1)
    fm2d = fm.reshape(_NPAD // 128, 128)
    src = jnp.pad(edge_index[0], (0, _EPAD - _E)).reshape(_NW, _NGRP, _GCH, _K)
    dst_flat = jnp.pad(edge_index[1], (0, _EPAD - _E), constant_values=_N)
    dst = dst_flat.reshape(_NW, _NGRP, _GCH, _K)
    b1r = b1.reshape(1, _D)
    b2r = b2.reshape(1, _D)
    use = pl.pallas_call(
        _any_body,
        out_shape=jax.ShapeDtypeStruct((1, 1), f32),
    )(fm2d)

    xz = pl.pallas_call(
        _prep_body,
        grid=(_GRID,),
        in_specs=[pl.BlockSpec((_RB, _D), lambda i: (i, 0)),
                  pl.BlockSpec((_RB, 1), lambda i: (i, 0))],
        out_specs=pl.BlockSpec((_RB, _D), lambda i: (i, 0)),
        out_shape=jax.ShapeDtypeStruct((_NPAD, _D), f32),
    )(x_p, fm_col)

    hist = pl.pallas_call(
        _hist_body,
        grid=(_HG,),
        in_specs=[pl.BlockSpec((_HR, _K), lambda i: (i, 0))],
        out_specs=pl.BlockSpec((_K, _K), lambda i: (0, 0)),
        out_shape=jax.ShapeDtypeStruct((_K, _K), f32),
    )(dst_flat.reshape(_EPAD // _K, _K))
    cnt_col = hist.reshape(-1)[:_NPAD].reshape(_NPAD, 1)

    part1 = _seg_sum(xz, src, dst)

    h1, cnt = pl.pallas_call(
        _layer1_body,
        grid=(_GRID,),
        in_specs=[
            pl.BlockSpec((_RB, _D), lambda i: (i, 0)),
            pl.BlockSpec((2, _RB, _D), lambda i: (0, i, 0)),
            pl.BlockSpec((_RB, 1), lambda i: (i, 0)),
            pl.BlockSpec((_RB, _D), lambda i: (i, 0)),
            pl.BlockSpec((_RB, 1), lambda i: (i, 0)),
            pl.BlockSpec((1, 1), lambda i: (0, 0)),
            pl.BlockSpec((_D, _D), lambda i: (0, 0)),
            pl.BlockSpec((_D, _D), lambda i: (0, 0)),
            pl.BlockSpec((1, _D), lambda i: (0, 0)),
        ],
        out_specs=[pl.BlockSpec((_RB, _D), lambda i: (i, 0)),
                   pl.BlockSpec((_RB, 1), lambda i: (i, 0))],
        out_shape=[jax.ShapeDtypeStruct((_NPAD, _D), f32),
                   jax.ShapeDtypeStruct((_NPAD, 1), f32)],
    )(xz, part1, cnt_col, agg_p, tm_col, use, W1_self, W1_neigh, b1r)

    part2 = _seg_sum(h1, src, dst)

    out = pl.pallas_call(
        _layer2_body,
        grid=(_GRID,),
        in_specs=[
            pl.BlockSpec((_RB, _D), lambda i: (i, 0)),
            pl.BlockSpec((2, _RB, _D), lambda i: (0, i, 0)),
            pl.BlockSpec((_RB, 1), lambda i: (i, 0)),
            pl.BlockSpec((_D, _D), lambda i: (0, 0)),
            pl.BlockSpec((_D, _D), lambda i: (0, 0)),
            pl.BlockSpec((1, _D), lambda i: (0, 0)),
        ],
        out_specs=pl.BlockSpec((_RB, _D), lambda i: (i, 0)),
        out_shape=jax.ShapeDtypeStruct((_NPAD, _D), f32),
    )(h1, part2, cnt, W2_self, W2_neigh, b2r)

    return out[:_N]
